# baseline (speedup 1.0000x reference)
import functools

import jax
import jax.numpy as jnp
from jax import lax
from jax.experimental import pallas as pl
from jax.experimental.pallas import tpu as pltpu
from jax.experimental.pallas import tpu_sc as plsc

_NUM_CORES = 2
_NUM_SUBCORES = 16
_NUM_WORKERS = _NUM_CORES * _NUM_SUBCORES
_CHUNK = 32
_NSLOTS = 3
_SC_BATCH = 1
_TC_BATCH = 3
_TC_BLK = 512


def _sc_bcast(pos_embedding, n, e):
    rows_per_w = n // _NUM_WORKERS
    nchunks = rows_per_w // _CHUNK
    mesh = plsc.VectorSubcoreMesh(
        core_axis_name="c", subcore_axis_name="s",
        num_cores=_NUM_CORES, num_subcores=_NUM_SUBCORES,
    )

    @functools.partial(
        pl.kernel,
        out_type=jax.ShapeDtypeStruct((_SC_BATCH, n, e), pos_embedding.dtype),
        mesh=mesh,
        scratch_types=[
            [pltpu.VMEM((_CHUNK, e), jnp.float32) for _ in range(_NSLOTS)],
            [pltpu.SemaphoreType.DMA for _ in range(_NSLOTS)],
            [pltpu.SemaphoreType.DMA for _ in range(_NSLOTS)],
        ],
    )
    def bcast(table_hbm, out_hbm, bufs, rsems, wsems):
        wid = lax.axis_index("s") * _NUM_CORES + lax.axis_index("c")
        base = wid * rows_per_w

        def read(g, slot):
            return pltpu.make_async_copy(
                table_hbm.at[pl.ds(base + g * _CHUNK, _CHUNK)],
                bufs[slot], rsems[slot])

        def writes(g, slot):
            return [
                pltpu.make_async_copy(
                    bufs[slot],
                    out_hbm.at[b, pl.ds(base + g * _CHUNK, _CHUNK)],
                    wsems[slot])
                for b in range(_SC_BATCH)
            ]

        for g in range(min(_NSLOTS, nchunks)):
            read(g, g % _NSLOTS).start()
        for g in range(nchunks):
            s = g % _NSLOTS
            read(g, s).wait()
            for w in writes(g, s):
                w.start()
            nxt = g + _NSLOTS
            if nxt < nchunks:
                for w in writes(g, s):
                    w.wait()
                read(nxt, s).start()
        for g in range(max(0, nchunks - _NSLOTS), nchunks):
            for w in writes(g, g % _NSLOTS):
                w.wait()

    return bcast(pos_embedding)


def _tc_body(table_ref, out_ref):
    blk = table_ref[...]
    out_ref[...] = jnp.broadcast_to(blk[None, :, :], (_TC_BATCH,) + blk.shape)


def _tc_bcast(pos_embedding, n, e):
    return pl.pallas_call(
        _tc_body,
        grid=(n // _TC_BLK,),
        in_specs=[pl.BlockSpec((_TC_BLK, e), lambda i: (i, 0))],
        out_specs=pl.BlockSpec((_TC_BATCH, _TC_BLK, e), lambda i: (0, i, 0)),
        out_shape=jax.ShapeDtypeStruct((_TC_BATCH, n, e), pos_embedding.dtype),
    )(pos_embedding)


def kernel(batch_size, seqlen, pos_embedding):
    n, e = pos_embedding.shape
    sc_part = _sc_bcast(pos_embedding, n, e)
    tc_part = _tc_bcast(pos_embedding, n, e)
    return jnp.concatenate([tc_part, sc_part], axis=0)


# speedup vs baseline: 2.1748x; 2.1748x over previous
import functools

import jax
import jax.numpy as jnp
from jax import lax
from jax.experimental import pallas as pl
from jax.experimental.pallas import tpu as pltpu
from jax.experimental.pallas import tpu_sc as plsc

_BATCH = 4
_NUM_CORES = 2
_NUM_SUBCORES = 16
_NUM_WORKERS = _NUM_CORES * _NUM_SUBCORES
_CHUNK = 32
_NSLOTS = 3


def kernel(batch_size, seqlen, pos_embedding):
    n, e = pos_embedding.shape
    rows_per_w = n // _NUM_WORKERS
    nchunks = rows_per_w // _CHUNK
    mesh = plsc.VectorSubcoreMesh(
        core_axis_name="c", subcore_axis_name="s",
        num_cores=_NUM_CORES, num_subcores=_NUM_SUBCORES,
    )

    @functools.partial(
        pl.kernel,
        out_type=jax.ShapeDtypeStruct((_BATCH, n, e), pos_embedding.dtype),
        mesh=mesh,
        scratch_types=[
            [pltpu.VMEM((_CHUNK, e), jnp.float32) for _ in range(_NSLOTS)],
            [pltpu.SemaphoreType.DMA for _ in range(_NSLOTS)],
            [pltpu.SemaphoreType.DMA for _ in range(_NSLOTS)],
        ],
    )
    def bcast(table_hbm, out_hbm, bufs, rsems, wsems):
        wid = lax.axis_index("s") * _NUM_CORES + lax.axis_index("c")
        base = wid * rows_per_w

        def read(g, slot):
            return pltpu.make_async_copy(
                table_hbm.at[pl.ds(base + g * _CHUNK, _CHUNK)],
                bufs[slot], rsems[slot])

        def writes(g, slot):
            return [
                pltpu.make_async_copy(
                    bufs[slot],
                    out_hbm.at[b, pl.ds(base + g * _CHUNK, _CHUNK)],
                    wsems[slot])
                for b in range(_BATCH)
            ]

        for g in range(min(_NSLOTS, nchunks)):
            read(g, g % _NSLOTS).start()
        for g in range(nchunks):
            s = g % _NSLOTS
            read(g, s).wait()
            for w in writes(g, s):
                w.start()
            nxt = g + _NSLOTS
            if nxt < nchunks:
                for w in writes(g, s):
                    w.wait()
                read(nxt, s).start()
        for g in range(max(0, nchunks - _NSLOTS), nchunks):
            for w in writes(g, g % _NSLOTS):
                w.wait()

    return bcast(pos_embedding)
